# Initial kernel scaffold; baseline (speedup 1.0000x reference)
import jax
import jax.numpy as jnp
from jax.experimental import pallas as pl


def kernel(idx, val, embedding):
    raise NotImplementedError("write your pallas kernel here")


# speedup vs baseline: 1.8390x; 1.8390x over previous
import functools

import jax
import jax.numpy as jnp
from jax import lax
from jax.experimental import pallas as pl
from jax.experimental.pallas import tpu as pltpu
from jax.experimental.pallas import tpu_sc as plsc

B = 4096
L = 50
D = 64
LP = 56
NW = 32
RPW = B // NW
NBUF = 2


def _body(idx_hbm, val_hbm, emb_hbm, out_hbm,
          idx_slab, val_slab, out_v, buf0, buf1, sem0, sem1):
    w = lax.axis_index("s") * 2 + lax.axis_index("c")
    base = w * RPW

    pltpu.sync_copy(idx_hbm.at[pl.ds(base * LP, RPW * LP)], idx_slab)
    pltpu.sync_copy(val_hbm.at[pl.ds(base * LP, RPW * LP)], val_slab)

    bufs = (buf0, buf1)
    sems = (sem0, sem1)

    for b in range(NBUF):
        pltpu.async_copy(
            emb_hbm.at[idx_slab.at[pl.ds(b * LP, LP)]], bufs[b], sems[b])

    def step(c, carry):
        for b in range(NBUF):
            row = c * NBUF + b
            pltpu.make_async_copy(
                emb_hbm.at[idx_slab.at[pl.ds(row * LP, LP)]],
                bufs[b], sems[b]).wait()
            accs = [jnp.zeros((16,), jnp.float32) for _ in range(4)]
            for l in range(LP):
                t = row * LP + l
                wv = plsc.load_gather(
                    val_slab, [jnp.full((16,), t, jnp.int32)])
                for k in range(4):
                    accs[k] = accs[k] + bufs[b][l, pl.ds(k * 16, 16)] * wv
            for k in range(4):
                out_v[row, pl.ds(k * 16, 16)] = accs[k]
            nxt = row + NBUF

            @pl.when(nxt < RPW)
            def _():
                pltpu.async_copy(
                    emb_hbm.at[idx_slab.at[pl.ds(nxt * LP, LP)]],
                    bufs[b], sems[b])
        return carry

    lax.fori_loop(0, RPW // NBUF, step, 0)

    pltpu.sync_copy(out_v, out_hbm.at[pl.ds(base, RPW), :])


@functools.partial(jax.jit, static_argnames=())
def _lookup(idx_flat, val_flat, embedding):
    mesh = plsc.VectorSubcoreMesh(core_axis_name="c", subcore_axis_name="s")
    return pl.kernel(
        _body,
        out_type=jax.ShapeDtypeStruct((B, D), jnp.float32),
        mesh=mesh,
        compiler_params=pltpu.CompilerParams(
            needs_layout_passes=False, use_tc_tiling_on_sc=False),
        scratch_types=[
            pltpu.VMEM((RPW * LP,), jnp.int32),
            pltpu.VMEM((RPW * LP,), jnp.float32),
            pltpu.VMEM((RPW, D), jnp.float32),
            pltpu.VMEM((LP, D), jnp.float32),
            pltpu.VMEM((LP, D), jnp.float32),
            pltpu.SemaphoreType.DMA,
            pltpu.SemaphoreType.DMA,
        ],
    )(idx_flat, val_flat, embedding)


def kernel(idx, val, embedding):
    idx_p = jnp.pad(idx.astype(jnp.int32), ((0, 0), (0, LP - L)))
    val_p = jnp.pad(val.astype(jnp.float32), ((0, 0), (0, LP - L)))
    out = _lookup(idx_p.reshape(-1), val_p.reshape(-1), embedding)
    return out[:, None, :]
